# baseline (speedup 1.0000x reference)
import jax
import jax.numpy as jnp
from jax.experimental import pallas as pl
from jax.experimental.pallas import tpu as pltpu

PS = 5
K = 14
H = 96
W = 96
C = 3
PDIM = C * PS * PS
NH = H - PS + 1
NW = W - PS + 1
N = NH * NW
LANES = 128
NPAD = ((N + LANES - 1) // LANES) * LANES
ROW_TILE = 128
GRID = NPAD // ROW_TILE


def _knn_agg_body(p_tile, p_full, pw_row, beta_arr, out_ref, pts):
    i = pl.program_id(0)

    @pl.when(i == 0)
    def _build_pts():
        for b in range(GRID):
            pts[:, b * LANES:(b + 1) * LANES] = p_full[b * LANES:(b + 1) * LANES, :].T

    beta = beta_arr[0:1, 0:1]
    pw = pw_row[0:1, :]

    lane = jax.lax.broadcasted_iota(jnp.int32, (ROW_TILE, LANES), 1)
    q = jnp.where(lane < PDIM, p_tile[...] * jnp.bfloat16(-2.0),
                  jnp.bfloat16(0.0))

    s = jax.lax.dot(q, pts[...], preferred_element_type=jnp.float32)
    s = s + pts[126:127, :].astype(jnp.float32)

    m = s[:, 0:LANES]
    for b in range(1, NPAD // LANES):
        m = jnp.minimum(m, s[:, b * LANES:(b + 1) * LANES])
    m1 = jnp.min(m, axis=1, keepdims=True)
    tau = m1
    for _ in range(K - 1):
        tau = jnp.min(jnp.where(m > tau, m, jnp.inf), axis=1, keepdims=True)

    w = jnp.where(s <= tau, jnp.exp2(beta * (m1 - s)), 0.0)

    d = jax.lax.dot(w.astype(jnp.bfloat16), p_full[...],
                    preferred_element_type=jnp.float32)
    norm = d[:, 127:128]
    out_ref[...] = (d * (pw / norm)).T


def _fold_body(dp_ref, pwb_ref, means_ref, out_ref, acc, den):
    acc[...] = jnp.zeros((C, H, LANES), dtype=jnp.float32)
    den[...] = jnp.zeros((C, H, LANES), dtype=jnp.float32)
    for c in range(C):
        for di in range(PS):
            for dj in range(PS):
                e = c * PS * PS + di * PS + dj
                acc[c, di:di + NH, dj:dj + NW] += dp_ref[e, :, :]
                den[c, di:di + NH, dj:dj + NW] += pwb_ref[e, 0:1, 0:1]
    out_ref[...] = (acc[...] / den[...] + means_ref[:, 0:1, :]) * 0.5 + 0.5


def _extract_patches_aug(x, sq_pad_value=1e30):
    parts = []
    for di in range(PS):
        for dj in range(PS):
            parts.append(x[:, di:di + NH, dj:dj + NW])
    p = jnp.stack(parts, axis=0)
    p = p.transpose(2, 3, 1, 0).reshape(N, PDIM)
    sq = jnp.sum(p * p, axis=1)
    p = jnp.pad(p, ((0, NPAD - N), (0, 0)))
    sq = jnp.pad(sq, (0, NPAD - N), constant_values=sq_pad_value)
    return jnp.concatenate(
        [p,
         jnp.zeros((NPAD, LANES - PDIM - 2), jnp.float32),
         sq[:, None],
         jnp.ones((NPAD, 1), jnp.float32)], axis=1).astype(jnp.bfloat16)


@jax.jit
def kernel(noisy, pw, beta):
    x = (noisy - 0.5) / 0.5
    means = x.mean(axis=(-2, -1), keepdims=True)
    x = (x - means)[0]

    p = _extract_patches_aug(x)
    pw_pad = jnp.pad(pw, (0, LANES - PDIM))
    pw_row = jnp.broadcast_to(pw_pad[None, :], (8, LANES))
    beta_arr = jnp.full((8, LANES), beta * 1.4426950408889634,
                        dtype=jnp.float32)

    deno_t = pl.pallas_call(
        _knn_agg_body,
        grid=(GRID,),
        in_specs=[
            pl.BlockSpec((ROW_TILE, LANES), lambda i: (i, 0)),
            pl.BlockSpec((NPAD, LANES), lambda i: (0, 0)),
            pl.BlockSpec((8, LANES), lambda i: (0, 0)),
            pl.BlockSpec((8, LANES), lambda i: (0, 0)),
        ],
        out_specs=pl.BlockSpec((LANES, ROW_TILE), lambda i: (0, i)),
        out_shape=jax.ShapeDtypeStruct((LANES, NPAD), jnp.float32),
        scratch_shapes=[pltpu.VMEM((LANES, NPAD), jnp.bfloat16)],
    )(p, p, pw_row, beta_arr)

    dp = deno_t[:PDIM, :N].reshape(PDIM, NH, NW)
    pw_bc = jnp.broadcast_to(pw[:, None, None], (PDIM, 8, LANES))
    means_in = jnp.broadcast_to(means[0, :, :, 0:1], (C, 8, LANES))

    img = pl.pallas_call(
        _fold_body,
        out_shape=jax.ShapeDtypeStruct((C, H, LANES), jnp.float32),
        scratch_shapes=[pltpu.VMEM((C, H, LANES), jnp.float32),
                        pltpu.VMEM((C, H, LANES), jnp.float32)],
    )(dp, pw_bc, means_in)

    return img[None, :, :, :W]


# speedup vs baseline: 1.0343x; 1.0343x over previous
import jax
import jax.numpy as jnp
from jax.experimental import pallas as pl
from jax.experimental.pallas import tpu as pltpu

PS = 5
K = 14
H = 96
W = 96
C = 3
PDIM = C * PS * PS
NH = H - PS + 1
NW = W - PS + 1
N = NH * NW
LANES = 128
NPAD = ((N + LANES - 1) // LANES) * LANES
ROW_TILE = 128
GRID = NPAD // ROW_TILE


def _knn_agg_body(p_tile, p_full, pw_row, beta_arr, out_ref, pts):
    i = pl.program_id(0)

    @pl.when(i == 0)
    def _build_pts():
        for b in range(GRID):
            pts[:, b * LANES:(b + 1) * LANES] = p_full[b * LANES:(b + 1) * LANES, :].T

    beta = beta_arr[0:1, 0:1]
    pw = pw_row[0:1, :]

    lane = jax.lax.broadcasted_iota(jnp.int32, (ROW_TILE, LANES), 1)
    q = jnp.where(lane < PDIM, p_tile[...] * jnp.bfloat16(-2.0),
                  jnp.bfloat16(0.0))

    s = jax.lax.dot(q, pts[...], preferred_element_type=jnp.float32)
    s = s + pts[126:127, :].astype(jnp.float32)

    m = s[:, 0:LANES]
    for b in range(1, NPAD // LANES):
        m = jnp.minimum(m, s[:, b * LANES:(b + 1) * LANES])
    m1 = jnp.min(m, axis=1, keepdims=True)
    tau = m1
    for _ in range(K - 1):
        tau = jnp.min(jnp.where(m > tau, m, jnp.inf), axis=1, keepdims=True)

    w = jnp.where(s <= tau, jnp.exp(beta * (m1 - s)), 0.0)

    d = jax.lax.dot(w.astype(jnp.bfloat16), p_full[...],
                    preferred_element_type=jnp.float32)
    norm = d[:, 127:128]
    out_ref[...] = (d * (pw / norm)).T


def _fold_body(dp_ref, pwb_ref, means_ref, out_ref, acc, den):
    acc[...] = jnp.zeros((C, H, LANES), dtype=jnp.float32)
    den[...] = jnp.zeros((C, H, LANES), dtype=jnp.float32)
    for c in range(C):
        for di in range(PS):
            for dj in range(PS):
                e = c * PS * PS + di * PS + dj
                acc[c, di:di + NH, dj:dj + NW] += dp_ref[e, :, :]
                den[c, di:di + NH, dj:dj + NW] += pwb_ref[e, 0:1, 0:1]
    out_ref[...] = (acc[...] / den[...] + means_ref[:, 0:1, :]) * 0.5 + 0.5


def _extract_patches_aug(x, sq_pad_value=1e30):
    parts = []
    for di in range(PS):
        for dj in range(PS):
            parts.append(x[:, di:di + NH, dj:dj + NW])
    p = jnp.stack(parts, axis=0)
    p = p.transpose(2, 3, 1, 0).reshape(N, PDIM)
    sq = jnp.sum(p * p, axis=1)
    p = jnp.pad(p, ((0, NPAD - N), (0, 0)))
    sq = jnp.pad(sq, (0, NPAD - N), constant_values=sq_pad_value)
    return jnp.concatenate(
        [p,
         jnp.zeros((NPAD, LANES - PDIM - 2), jnp.float32),
         sq[:, None],
         jnp.ones((NPAD, 1), jnp.float32)], axis=1).astype(jnp.bfloat16)


@jax.jit
def kernel(noisy, pw, beta):
    x = (noisy - 0.5) / 0.5
    means = x.mean(axis=(-2, -1), keepdims=True)
    x = (x - means)[0]

    p = _extract_patches_aug(x)
    pw_pad = jnp.pad(pw, (0, LANES - PDIM))
    pw_row = jnp.broadcast_to(pw_pad[None, :], (8, LANES))
    beta_arr = jnp.full((8, LANES), beta, dtype=jnp.float32)

    deno_t = pl.pallas_call(
        _knn_agg_body,
        grid=(GRID,),
        in_specs=[
            pl.BlockSpec((ROW_TILE, LANES), lambda i: (i, 0)),
            pl.BlockSpec((NPAD, LANES), lambda i: (0, 0)),
            pl.BlockSpec((8, LANES), lambda i: (0, 0)),
            pl.BlockSpec((8, LANES), lambda i: (0, 0)),
        ],
        out_specs=pl.BlockSpec((LANES, ROW_TILE), lambda i: (0, i)),
        out_shape=jax.ShapeDtypeStruct((LANES, NPAD), jnp.float32),
        scratch_shapes=[pltpu.VMEM((LANES, NPAD), jnp.bfloat16)],
    )(p, p, pw_row, beta_arr)

    dp = deno_t[:PDIM, :N].reshape(PDIM, NH, NW)
    pw_bc = jnp.broadcast_to(pw[:, None, None], (PDIM, 8, LANES))
    means_in = jnp.broadcast_to(means[0, :, :, 0:1], (C, 8, LANES))

    img = pl.pallas_call(
        _fold_body,
        out_shape=jax.ShapeDtypeStruct((C, H, LANES), jnp.float32),
        scratch_shapes=[pltpu.VMEM((C, H, LANES), jnp.float32),
                        pltpu.VMEM((C, H, LANES), jnp.float32)],
    )(dp, pw_bc, means_in)

    return img[None, :, :, :W]


# speedup vs baseline: 1.2830x; 1.2405x over previous
import jax
import jax.numpy as jnp
from jax.experimental import pallas as pl
from jax.experimental.pallas import tpu as pltpu

PS = 5
K = 14
H = 96
W = 96
C = 3
PDIM = C * PS * PS
NH = H - PS + 1
NW = W - PS + 1
N = NH * NW
LANES = 128
ROW_TILE = 256
NPAD = ((N + ROW_TILE - 1) // ROW_TILE) * ROW_TILE
GRID = NPAD // ROW_TILE
LBLOCKS = NPAD // LANES


def _knn_agg_body(p_tile, p_full, pw_row, beta_arr, out_ref, pts):
    i = pl.program_id(0)

    @pl.when(i == 0)
    def _build_pts():
        for b in range(LBLOCKS):
            pts[:, b * LANES:(b + 1) * LANES] = p_full[b * LANES:(b + 1) * LANES, :].T

    beta = beta_arr[0:1, 0:1]
    pw = pw_row[0:1, :]

    lane = jax.lax.broadcasted_iota(jnp.int32, (ROW_TILE, LANES), 1)
    q = jnp.where(lane < PDIM, p_tile[...] * jnp.bfloat16(-2.0),
                  jnp.bfloat16(0.0))

    s = jax.lax.dot(q, pts[...], preferred_element_type=jnp.float32)
    s = s + pts[126:127, :].astype(jnp.float32)

    m = s[:, 0:LANES]
    for b in range(1, LBLOCKS):
        m = jnp.minimum(m, s[:, b * LANES:(b + 1) * LANES])
    m1 = jnp.min(m, axis=1, keepdims=True)
    tau = m1
    for _ in range(K - 1):
        tau = jnp.min(jnp.where(m > tau, m, jnp.inf), axis=1, keepdims=True)

    w = jnp.where(s <= tau, jnp.exp(beta * (m1 - s)), 0.0)

    d = jax.lax.dot(w.astype(jnp.bfloat16), p_full[...],
                    preferred_element_type=jnp.float32)
    norm = d[:, 127:128]
    out_ref[...] = (d * (pw / norm)).T


def _fold_body(dp_ref, pwb_ref, means_ref, out_ref, acc, den):
    acc[...] = jnp.zeros((C, H, LANES), dtype=jnp.float32)
    den[...] = jnp.zeros((C, H, LANES), dtype=jnp.float32)
    for c in range(C):
        for di in range(PS):
            for dj in range(PS):
                e = c * PS * PS + di * PS + dj
                acc[c, di:di + NH, dj:dj + NW] += dp_ref[e, :, :]
                den[c, di:di + NH, dj:dj + NW] += pwb_ref[e, 0:1, 0:1]
    out_ref[...] = (acc[...] / den[...] + means_ref[:, 0:1, :]) * 0.5 + 0.5


def _extract_patches_aug(x, sq_pad_value=1e30):
    parts = []
    for di in range(PS):
        for dj in range(PS):
            parts.append(x[:, di:di + NH, dj:dj + NW])
    p = jnp.stack(parts, axis=0)
    p = p.transpose(2, 3, 1, 0).reshape(N, PDIM)
    sq = jnp.sum(p * p, axis=1)
    p = jnp.pad(p, ((0, NPAD - N), (0, 0)))
    sq = jnp.pad(sq, (0, NPAD - N), constant_values=sq_pad_value)
    return jnp.concatenate(
        [p,
         jnp.zeros((NPAD, LANES - PDIM - 2), jnp.float32),
         sq[:, None],
         jnp.ones((NPAD, 1), jnp.float32)], axis=1).astype(jnp.bfloat16)


@jax.jit
def kernel(noisy, pw, beta):
    x = (noisy - 0.5) / 0.5
    means = x.mean(axis=(-2, -1), keepdims=True)
    x = (x - means)[0]

    p = _extract_patches_aug(x)
    pw_pad = jnp.pad(pw, (0, LANES - PDIM))
    pw_row = jnp.broadcast_to(pw_pad[None, :], (8, LANES))
    beta_arr = jnp.full((8, LANES), beta, dtype=jnp.float32)

    deno_t = pl.pallas_call(
        _knn_agg_body,
        grid=(GRID,),
        in_specs=[
            pl.BlockSpec((ROW_TILE, LANES), lambda i: (i, 0)),
            pl.BlockSpec((NPAD, LANES), lambda i: (0, 0)),
            pl.BlockSpec((8, LANES), lambda i: (0, 0)),
            pl.BlockSpec((8, LANES), lambda i: (0, 0)),
        ],
        out_specs=pl.BlockSpec((LANES, ROW_TILE), lambda i: (0, i)),
        out_shape=jax.ShapeDtypeStruct((LANES, NPAD), jnp.float32),
        scratch_shapes=[pltpu.VMEM((LANES, NPAD), jnp.bfloat16)],
    )(p, p, pw_row, beta_arr)

    dp = deno_t[:PDIM, :N].reshape(PDIM, NH, NW)
    pw_bc = jnp.broadcast_to(pw[:, None, None], (PDIM, 8, LANES))
    means_in = jnp.broadcast_to(means[0, :, :, 0:1], (C, 8, LANES))

    img = pl.pallas_call(
        _fold_body,
        out_shape=jax.ShapeDtypeStruct((C, H, LANES), jnp.float32),
        scratch_shapes=[pltpu.VMEM((C, H, LANES), jnp.float32),
                        pltpu.VMEM((C, H, LANES), jnp.float32)],
    )(dp, pw_bc, means_in)

    return img[None, :, :, :W]
